# Initial kernel scaffold; baseline (speedup 1.0000x reference)
import jax
import jax.numpy as jnp
from jax.experimental import pallas as pl


def kernel(node_feature, edge_index, edge_feature, params):
    raise NotImplementedError("write your pallas kernel here")


# speedup vs baseline: 2.2647x; 2.2647x over previous
import functools

import jax
import jax.numpy as jnp
from jax import lax
from jax.experimental import pallas as pl
from jax.experimental.pallas import tpu as pltpu
from jax.experimental.pallas import tpu_sc as plsc

_N = 10000
_E = 320000
_IN = 128
_ED = 20
_NH = 32
_FF = 32
_LAYERS = 3

_NW = 32
_PW = _E // _NW
_CH = 1024
_NCHUNK = -(-_PW // _CH)
_PWP = _NCHUNK * _CH
_EP = _NW * _PWP
_NP = 10016
_RPS = _NP // 16
_F32 = jnp.float32


def _dot(a, b):
    return jnp.dot(a, b, preferred_element_type=_F32)


def _node_pre_body(x_ref, w0, b0, w1, b1, lw, lb, meta_ref, res_ref):
    x = x_ref[...]
    h = jnp.maximum(_dot(x, w0[...]) + b0[...], 0.0)
    res = _dot(h, w1[...]) + b1[...]
    res_ref[...] = res
    meta_ref[...] = _dot(x, lw[...]) + lb[...] + res


def _edge_pre_body(x_ref, w0, b0, w1, b1, lw, lb, meta_ref):
    x = x_ref[...]
    h = jnp.maximum(_dot(x, w0[...]) + b0[...], 0.0)
    res = _dot(h, w1[...]) + b1[...]
    meta_ref[...] = _dot(x, lw[...]) + lb[...] + res


def _edge_layer_body(g_ref, me_ref, w0a, w0b, b0, w1, b1, h_ref):
    t = _dot(g_ref[...], w0a[...]) + _dot(me_ref[...], w0b[...]) + b0[...]
    t = jnp.maximum(t, 0.0)
    h_ref[...] = _dot(t, w1[...]) + b1[...]


def _node_layer_body(x_ref, part_ref, cnt_ref, res_ref, w0, b0, w1, b1, out_ref):
    agg = part_ref[0] + part_ref[1]
    cnt = cnt_ref[0] + cnt_ref[1]
    agg = agg / jnp.maximum(cnt, 1.0)
    out = jnp.maximum(x_ref[...] + agg, 0.0)
    h = jnp.maximum(_dot(out, w0[...]) + b0[...], 0.0)
    out_ref[...] = out + _dot(h, w1[...]) + b1[...] + res_ref[...]


def _full_spec(shape):
    return pl.BlockSpec(shape, lambda i: (0,) * len(shape))


_MESH = plsc.VectorSubcoreMesh(core_axis_name="c", subcore_axis_name="s")
_SC_PARAMS = pltpu.CompilerParams(use_tc_tiling_on_sc=False)


@functools.partial(
    pl.kernel,
    mesh=_MESH,
    out_type=jax.ShapeDtypeStruct((_EP, _NH), _F32),
    compiler_params=_SC_PARAMS,
    scratch_types=[
        pltpu.VMEM((8, 128), jnp.int32),
        pltpu.VMEM((_CH, _NH), _F32),
        pltpu.SemaphoreType.DMA,
    ],
)
def _sc_gather(table_hbm, idx_hbm, out_hbm, idx_v, rows_v, sem):
    c = lax.axis_index("c")
    s = lax.axis_index("s")
    wid = s * 2 + c

    def body(t, carry):
        base = wid * _PWP + t * _CH
        row0 = wid * (_PWP // 128) + t * 8
        pltpu.sync_copy(idx_hbm.at[pl.ds(row0, 8)], idx_v)
        descs = [
            pltpu.async_copy(
                table_hbm.at[idx_v.at[j]],
                rows_v.at[pl.ds(j * 128, 128)],
                sem,
            )
            for j in range(8)
        ]
        for d in descs:
            d.wait()
        pltpu.sync_copy(rows_v, out_hbm.at[pl.ds(base, _CH)])
        return carry

    lax.fori_loop(0, _NCHUNK, body, 0)


@functools.partial(
    pl.kernel,
    mesh=_MESH,
    out_type=jax.ShapeDtypeStruct((2, _NP, _NH), _F32),
    compiler_params=_SC_PARAMS,
    scratch_types=[
        pltpu.VMEM((8, 128), jnp.int32),
        pltpu.VMEM((_CH, _NH), _F32),
        pltpu.VMEM_SHARED((_NP, _NH), _F32),
        pltpu.SemaphoreType.DMA,
    ],
)
def _sc_scatter(vals_hbm, idx_hbm, zeros_hbm, out_hbm, idx_v, rows_v, acc_sh, sem):
    c = lax.axis_index("c")
    s = lax.axis_index("s")
    wid = s * 2 + c

    pltpu.sync_copy(zeros_hbm, acc_sh.at[pl.ds(s * _RPS, _RPS)])
    plsc.subcore_barrier()

    def body(t, carry):
        base = wid * _PWP + t * _CH
        row0 = wid * (_PWP // 128) + t * 8
        pltpu.sync_copy(idx_hbm.at[pl.ds(row0, 8)], idx_v)
        pltpu.sync_copy(vals_hbm.at[pl.ds(base, _CH)], rows_v)
        for j in range(8):
            pltpu.sync_copy(
                rows_v.at[pl.ds(j * 128, 128)],
                acc_sh.at[idx_v.at[j]],
                add=True,
            )
        return carry

    lax.fori_loop(0, _NCHUNK, body, 0)
    plsc.subcore_barrier()
    pltpu.sync_copy(
        acc_sh.at[pl.ds(s * _RPS, _RPS)],
        out_hbm.at[c, pl.ds(s * _RPS, _RPS)],
    )


@functools.partial(
    pl.kernel,
    mesh=_MESH,
    out_type=jax.ShapeDtypeStruct((2, _NP, _NH), _F32),
    compiler_params=_SC_PARAMS,
    scratch_types=[
        pltpu.VMEM((8, 128), jnp.int32),
        pltpu.VMEM((128, _NH), _F32),
        pltpu.VMEM_SHARED((_NP, _NH), _F32),
        pltpu.SemaphoreType.DMA,
    ],
)
def _sc_counts(idx_hbm, zeros_hbm, ones_hbm, out_hbm, idx_v, ones_v, acc_sh, sem):
    c = lax.axis_index("c")
    s = lax.axis_index("s")
    wid = s * 2 + c

    pltpu.sync_copy(ones_hbm, ones_v)
    pltpu.sync_copy(zeros_hbm, acc_sh.at[pl.ds(s * _RPS, _RPS)])
    plsc.subcore_barrier()

    def body(t, carry):
        row0 = wid * (_PWP // 128) + t * 8
        pltpu.sync_copy(idx_hbm.at[pl.ds(row0, 8)], idx_v)
        for j in range(8):
            pltpu.sync_copy(ones_v, acc_sh.at[idx_v.at[j]], add=True)
        return carry

    lax.fori_loop(0, _NCHUNK, body, 0)
    plsc.subcore_barrier()
    pltpu.sync_copy(
        acc_sh.at[pl.ds(s * _RPS, _RPS)],
        out_hbm.at[c, pl.ds(s * _RPS, _RPS)],
    )


def kernel(node_feature, edge_index, edge_feature, params):
    p = params
    src = edge_index[0]
    dst = edge_index[1]
    pad = _PWP - _PW
    src_p = jnp.pad(src.reshape(_NW, _PW), ((0, 0), (0, pad)))
    dst_p = jnp.pad(dst.reshape(_NW, _PW), ((0, 0), (0, pad)), constant_values=_N)
    src2d = src_p.reshape(_EP // 128, 128)
    dst2d = dst_p.reshape(_EP // 128, 128)
    ef_p = jnp.pad(
        edge_feature.reshape(_NW, _PW, _ED), ((0, 0), (0, pad), (0, 0))
    ).reshape(_EP, _ED)

    zeros_sub = jnp.zeros((_RPS, _NH), _F32)
    ones128 = jnp.ones((128, _NH), _F32)

    def r2(b):
        return b.reshape(1, -1)

    nblk, ngrid = 2000, 5
    meta_node, node_res = pl.pallas_call(
        _node_pre_body,
        grid=(ngrid,),
        in_specs=[
            pl.BlockSpec((nblk, _IN), lambda i: (i, 0)),
            _full_spec((_IN, _FF)),
            _full_spec((1, _FF)),
            _full_spec((_FF, _NH)),
            _full_spec((1, _NH)),
            _full_spec((_IN, _NH)),
            _full_spec((1, _NH)),
        ],
        out_specs=[
            pl.BlockSpec((nblk, _NH), lambda i: (i, 0)),
            pl.BlockSpec((nblk, _NH), lambda i: (i, 0)),
        ],
        out_shape=[
            jax.ShapeDtypeStruct((_N, _NH), _F32),
            jax.ShapeDtypeStruct((_N, _NH), _F32),
        ],
    )(
        node_feature,
        p["node_ff_w0"], r2(p["node_ff_b0"]), p["node_ff_w1"], r2(p["node_ff_b1"]),
        p["node_lin_w"], r2(p["node_lin_b"]),
    )

    eblk, egrid = 4096, _EP // 4096
    meta_edge = pl.pallas_call(
        _edge_pre_body,
        grid=(egrid,),
        in_specs=[
            pl.BlockSpec((eblk, _ED), lambda i: (i, 0)),
            _full_spec((_ED, _FF)),
            _full_spec((1, _FF)),
            _full_spec((_FF, _NH)),
            _full_spec((1, _NH)),
            _full_spec((_ED, _NH)),
            _full_spec((1, _NH)),
        ],
        out_specs=pl.BlockSpec((eblk, _NH), lambda i: (i, 0)),
        out_shape=jax.ShapeDtypeStruct((_EP, _NH), _F32),
    )(
        ef_p,
        p["edge_ff_w0"], r2(p["edge_ff_b0"]), p["edge_ff_w1"], r2(p["edge_ff_b1"]),
        p["edge_lin_w"], r2(p["edge_lin_b"]),
    )

    cnt_part = _sc_counts(dst2d, zeros_sub, ones128)

    x = meta_node
    for l in range(_LAYERS):
        g = _sc_gather(x, src2d)
        h = pl.pallas_call(
            _edge_layer_body,
            grid=(egrid,),
            in_specs=[
                pl.BlockSpec((eblk, _NH), lambda i: (i, 0)),
                pl.BlockSpec((eblk, _NH), lambda i: (i, 0)),
                _full_spec((_NH, _FF)),
                _full_spec((_NH, _FF)),
                _full_spec((1, _FF)),
                _full_spec((_FF, _NH)),
                _full_spec((1, _NH)),
            ],
            out_specs=pl.BlockSpec((eblk, _NH), lambda i: (i, 0)),
            out_shape=jax.ShapeDtypeStruct((_EP, _NH), _F32),
        )(
            g, meta_edge,
            p[f"gc{l}_msg_w0"][:_NH], p[f"gc{l}_msg_w0"][_NH:],
            r2(p[f"gc{l}_msg_b0"]),
            p[f"gc{l}_msg_w1"], r2(p[f"gc{l}_msg_b1"]),
        )
        part = _sc_scatter(h, dst2d, zeros_sub)
        x = pl.pallas_call(
            _node_layer_body,
            grid=(ngrid,),
            in_specs=[
                pl.BlockSpec((nblk, _NH), lambda i: (i, 0)),
                pl.BlockSpec((2, nblk, _NH), lambda i: (0, i, 0)),
                pl.BlockSpec((2, nblk, _NH), lambda i: (0, i, 0)),
                pl.BlockSpec((nblk, _NH), lambda i: (i, 0)),
                _full_spec((_NH, _FF)),
                _full_spec((1, _FF)),
                _full_spec((_FF, _NH)),
                _full_spec((1, _NH)),
            ],
            out_specs=pl.BlockSpec((nblk, _NH), lambda i: (i, 0)),
            out_shape=jax.ShapeDtypeStruct((_N, _NH), _F32),
        )(
            x, part, cnt_part, node_res,
            p[f"gc{l}_res_w0"], r2(p[f"gc{l}_res_b0"]),
            p[f"gc{l}_res_w1"], r2(p[f"gc{l}_res_b1"]),
        )
    return x


# speedup vs baseline: 4.6564x; 2.0561x over previous
import functools

import jax
import jax.numpy as jnp
from jax import lax
from jax.experimental import pallas as pl
from jax.experimental.pallas import tpu as pltpu
from jax.experimental.pallas import tpu_sc as plsc

_N = 10000
_E = 320000
_IN = 128
_ED = 20
_NH = 32
_FF = 32
_LAYERS = 3

_NW = 32
_PW = _E // _NW
_CH = 1024
_NCHUNK = -(-_PW // _CH)
_PWP = _NCHUNK * _CH
_EP = _NW * _PWP
_NP = 10016
_RPS = _NP // 16
_F32 = jnp.float32


def _dot(a, b):
    return jnp.dot(a, b, preferred_element_type=_F32)


def _node_pre_body(x_ref, w0, b0, w1, b1, lw, lb, meta_ref, res_ref):
    x = x_ref[...]
    h = jnp.maximum(_dot(x, w0[...]) + b0[...], 0.0)
    res = _dot(h, w1[...]) + b1[...]
    res_ref[...] = res
    meta_ref[...] = _dot(x, lw[...]) + lb[...] + res


def _edge_pre_body(x_ref, w0, b0, w1, b1, lw, lb, meta_ref):
    x = x_ref[...]
    h = jnp.maximum(_dot(x, w0[...]) + b0[...], 0.0)
    res = _dot(h, w1[...]) + b1[...]
    meta_ref[...] = _dot(x, lw[...]) + lb[...] + res


def _edge_layer_body(g_ref, me_ref, w0a, w0b, b0, w1, b1, h_ref):
    t = _dot(g_ref[...], w0a[...]) + _dot(me_ref[...], w0b[...]) + b0[...]
    t = jnp.maximum(t, 0.0)
    h_ref[...] = _dot(t, w1[...]) + b1[...]


def _node_layer_body(x_ref, part_ref, cnt_ref, res_ref, w0, b0, w1, b1, out_ref):
    nr = _N // 4
    agg = part_ref[0, :nr] + part_ref[1, :nr]
    cnt = cnt_ref[0, :nr] + cnt_ref[1, :nr]
    agg = agg / jnp.maximum(cnt, 1.0)
    out = jnp.maximum(x_ref[...] + agg, 0.0)
    h = jnp.maximum(_dot(out, w0[...]) + b0[...], 0.0)
    out_ref[...] = out + _dot(h, w1[...]) + b1[...] + res_ref[...]


def _full_spec(shape):
    return pl.BlockSpec(shape, lambda i: (0,) * len(shape))


_MESH = plsc.VectorSubcoreMesh(core_axis_name="c", subcore_axis_name="s")
_SC_PARAMS = pltpu.CompilerParams(use_tc_tiling_on_sc=False)


@functools.partial(
    pl.kernel,
    mesh=_MESH,
    out_type=jax.ShapeDtypeStruct((_EP, _NH), _F32),
    compiler_params=_SC_PARAMS,
    scratch_types=[
        pltpu.VMEM((8, 128), jnp.int32),
        pltpu.VMEM((_CH, _NH), _F32),
        pltpu.SemaphoreType.DMA,
    ],
)
def _sc_gather(table_hbm, idx_hbm, out_hbm, idx_v, rows_v, sem):
    c = lax.axis_index("c")
    s = lax.axis_index("s")
    wid = s * 2 + c

    def body(t, carry):
        base = wid * _PWP + t * _CH
        row0 = wid * (_PWP // 128) + t * 8
        pltpu.sync_copy(idx_hbm.at[pl.ds(row0, 8)], idx_v)
        descs = [
            pltpu.async_copy(
                table_hbm.at[idx_v.at[j]],
                rows_v.at[pl.ds(j * 128, 128)],
                sem,
            )
            for j in range(8)
        ]
        for d in descs:
            d.wait()
        pltpu.sync_copy(rows_v, out_hbm.at[pl.ds(base, _CH)])
        return carry

    lax.fori_loop(0, _NCHUNK, body, 0)


@functools.partial(
    pl.kernel,
    mesh=_MESH,
    out_type=jax.ShapeDtypeStruct((2, _NP, _NH), _F32),
    compiler_params=_SC_PARAMS,
    scratch_types=[
        pltpu.VMEM((8, 128), jnp.int32),
        pltpu.VMEM((_CH, _NH), _F32),
        pltpu.VMEM_SHARED((_NP, _NH), _F32),
        pltpu.SemaphoreType.DMA,
    ],
)
def _sc_scatter(vals_hbm, idx_hbm, zeros_hbm, out_hbm, idx_v, rows_v, acc_sh, sem):
    c = lax.axis_index("c")
    s = lax.axis_index("s")
    wid = s * 2 + c

    pltpu.sync_copy(zeros_hbm, acc_sh.at[pl.ds(s * _RPS, _RPS)])
    plsc.subcore_barrier()

    def body(t, carry):
        base = wid * _PWP + t * _CH
        row0 = wid * (_PWP // 128) + t * 8
        pltpu.sync_copy(idx_hbm.at[pl.ds(row0, 8)], idx_v)
        pltpu.sync_copy(vals_hbm.at[pl.ds(base, _CH)], rows_v)
        for j in range(8):
            pltpu.sync_copy(
                rows_v.at[pl.ds(j * 128, 128)],
                acc_sh.at[idx_v.at[j]],
                add=True,
            )
        return carry

    lax.fori_loop(0, _NCHUNK, body, 0)
    plsc.subcore_barrier()
    pltpu.sync_copy(
        acc_sh.at[pl.ds(s * _RPS, _RPS)],
        out_hbm.at[c, pl.ds(s * _RPS, _RPS)],
    )


@functools.partial(
    pl.kernel,
    mesh=_MESH,
    out_type=jax.ShapeDtypeStruct((2, _NP, _NH), _F32),
    compiler_params=_SC_PARAMS,
    scratch_types=[
        pltpu.VMEM((8, 128), jnp.int32),
        pltpu.VMEM((128, _NH), _F32),
        pltpu.VMEM_SHARED((_NP, _NH), _F32),
        pltpu.SemaphoreType.DMA,
    ],
)
def _sc_counts(idx_hbm, zeros_hbm, ones_hbm, out_hbm, idx_v, ones_v, acc_sh, sem):
    c = lax.axis_index("c")
    s = lax.axis_index("s")
    wid = s * 2 + c

    pltpu.sync_copy(ones_hbm, ones_v)
    pltpu.sync_copy(zeros_hbm, acc_sh.at[pl.ds(s * _RPS, _RPS)])
    plsc.subcore_barrier()

    def body(t, carry):
        row0 = wid * (_PWP // 128) + t * 8
        pltpu.sync_copy(idx_hbm.at[pl.ds(row0, 8)], idx_v)
        for j in range(8):
            pltpu.sync_copy(ones_v, acc_sh.at[idx_v.at[j]], add=True)
        return carry

    lax.fori_loop(0, _NCHUNK, body, 0)
    plsc.subcore_barrier()
    pltpu.sync_copy(
        acc_sh.at[pl.ds(s * _RPS, _RPS)],
        out_hbm.at[c, pl.ds(s * _RPS, _RPS)],
    )


def kernel(node_feature, edge_index, edge_feature, params):
    p = params
    src = edge_index[0]
    dst = edge_index[1]
    pad = _PWP - _PW
    src_p = jnp.pad(src.reshape(_NW, _PW), ((0, 0), (0, pad)))
    dst_p = jnp.pad(dst.reshape(_NW, _PW), ((0, 0), (0, pad)), constant_values=_N)
    src2d = src_p.reshape(_EP // 128, 128)
    dst2d = dst_p.reshape(_EP // 128, 128)
    ef4 = jnp.pad(
        edge_feature.reshape(_NW, _PW, _ED), ((0, 0), (0, pad), (0, 12))
    ).reshape(_EP // 4, 128)

    zeros_sub = jnp.zeros((_RPS, _NH), _F32)
    ones128 = jnp.ones((128, _NH), _F32)

    eye4 = jnp.eye(4, dtype=_F32)

    def bd(w):
        return jnp.kron(eye4, w)

    def bt(b):
        return jnp.tile(b, 4).reshape(1, 4 * b.shape[0])

    nf4 = node_feature.reshape(_N // 4, 4 * _IN)
    nblk = _N // 4
    meta_node4, node_res4 = pl.pallas_call(
        _node_pre_body,
        grid=(1,),
        in_specs=[
            pl.BlockSpec((nblk, 4 * _IN), lambda i: (i, 0)),
            _full_spec((4 * _IN, 4 * _FF)),
            _full_spec((1, 4 * _FF)),
            _full_spec((4 * _FF, 4 * _NH)),
            _full_spec((1, 4 * _NH)),
            _full_spec((4 * _IN, 4 * _NH)),
            _full_spec((1, 4 * _NH)),
        ],
        out_specs=[
            pl.BlockSpec((nblk, 4 * _NH), lambda i: (i, 0)),
            pl.BlockSpec((nblk, 4 * _NH), lambda i: (i, 0)),
        ],
        out_shape=[
            jax.ShapeDtypeStruct((_N // 4, 4 * _NH), _F32),
            jax.ShapeDtypeStruct((_N // 4, 4 * _NH), _F32),
        ],
    )(
        nf4,
        bd(p["node_ff_w0"]), bt(p["node_ff_b0"]),
        bd(p["node_ff_w1"]), bt(p["node_ff_b1"]),
        bd(p["node_lin_w"]), bt(p["node_lin_b"]),
    )

    eblk, egrid = 4096, _EP // 4 // 4096
    ew0 = jnp.pad(p["edge_ff_w0"], ((0, 12), (0, 0)))
    elw = jnp.pad(p["edge_lin_w"], ((0, 12), (0, 0)))
    meta_edge4 = pl.pallas_call(
        _edge_pre_body,
        grid=(egrid,),
        in_specs=[
            pl.BlockSpec((eblk, 128), lambda i: (i, 0)),
            _full_spec((128, 4 * _FF)),
            _full_spec((1, 4 * _FF)),
            _full_spec((4 * _FF, 4 * _NH)),
            _full_spec((1, 4 * _NH)),
            _full_spec((128, 4 * _NH)),
            _full_spec((1, 4 * _NH)),
        ],
        out_specs=pl.BlockSpec((eblk, 4 * _NH), lambda i: (i, 0)),
        out_shape=jax.ShapeDtypeStruct((_EP // 4, 4 * _NH), _F32),
    )(
        ef4,
        bd(ew0), bt(p["edge_ff_b0"]),
        bd(p["edge_ff_w1"]), bt(p["edge_ff_b1"]),
        bd(elw), bt(p["edge_lin_b"]),
    )

    cnt_part = _sc_counts(dst2d, zeros_sub, ones128)
    cnt4 = cnt_part.reshape(2, _NP // 4, 4 * _NH)

    x4 = meta_node4
    for l in range(_LAYERS):
        g = _sc_gather(x4.reshape(_N, _NH), src2d)
        h4 = pl.pallas_call(
            _edge_layer_body,
            grid=(egrid,),
            in_specs=[
                pl.BlockSpec((eblk, 4 * _NH), lambda i: (i, 0)),
                pl.BlockSpec((eblk, 4 * _NH), lambda i: (i, 0)),
                _full_spec((4 * _NH, 4 * _FF)),
                _full_spec((4 * _NH, 4 * _FF)),
                _full_spec((1, 4 * _FF)),
                _full_spec((4 * _FF, 4 * _NH)),
                _full_spec((1, 4 * _NH)),
            ],
            out_specs=pl.BlockSpec((eblk, 4 * _NH), lambda i: (i, 0)),
            out_shape=jax.ShapeDtypeStruct((_EP // 4, 4 * _NH), _F32),
        )(
            g.reshape(_EP // 4, 4 * _NH), meta_edge4,
            bd(p[f"gc{l}_msg_w0"][:_NH]), bd(p[f"gc{l}_msg_w0"][_NH:]),
            bt(p[f"gc{l}_msg_b0"]),
            bd(p[f"gc{l}_msg_w1"]), bt(p[f"gc{l}_msg_b1"]),
        )
        part = _sc_scatter(h4.reshape(_EP, _NH), dst2d, zeros_sub)
        x4 = pl.pallas_call(
            _node_layer_body,
            grid=(1,),
            in_specs=[
                pl.BlockSpec((_N // 4, 4 * _NH), lambda i: (0, 0)),
                pl.BlockSpec((2, _NP // 4, 4 * _NH), lambda i: (0, 0, 0)),
                pl.BlockSpec((2, _NP // 4, 4 * _NH), lambda i: (0, 0, 0)),
                pl.BlockSpec((_N // 4, 4 * _NH), lambda i: (0, 0)),
                _full_spec((4 * _NH, 4 * _FF)),
                _full_spec((1, 4 * _FF)),
                _full_spec((4 * _FF, 4 * _NH)),
                _full_spec((1, 4 * _NH)),
            ],
            out_specs=pl.BlockSpec((_N // 4, 4 * _NH), lambda i: (0, 0)),
            out_shape=jax.ShapeDtypeStruct((_N // 4, 4 * _NH), _F32),
        )(
            x4, part.reshape(2, _NP // 4, 4 * _NH), cnt4, node_res4,
            bd(p[f"gc{l}_res_w0"]), bt(p[f"gc{l}_res_b0"]),
            bd(p[f"gc{l}_res_w1"]), bt(p[f"gc{l}_res_b1"]),
        )
    return x4.reshape(_N, _NH)
